# baseline (speedup 1.0000x reference)
import numpy as np
import jax
import jax.numpy as jnp
from jax.experimental import pallas as pl

_NUM_RES = 190
_CYT = np.arange(0, _NUM_RES, 4)
_NCYT = len(_CYT)
_HID = 10
_NB = 4096
_NSPLIT = 4
_NC = _NB // _NSPLIT


def _body(*refs):
    d_refs = refs[:2 * _NSPLIT]
    (f_ref, A_ref, Bm_ref, m_ref, mcyt_ref, base_ref,
     slope_ref, b1_ref, w2a_ref, w2b_ref, b2_ref, out_ref) = refs[2 * _NSPLIT:]
    dn = (((1,), (0,)), ((), ()))
    chunks = []
    for k in range(_NSPLIT):
        d1 = d_refs[2 * k][0].astype(jnp.bfloat16)
        d2 = d_refs[2 * k + 1][0].astype(jnp.bfloat16)
        chunks.append(
            jax.lax.dot_general(A_ref[...], d1, dimension_numbers=dn,
                                preferred_element_type=jnp.float32)
            + jax.lax.dot_general(Bm_ref[...], d2, dimension_numbers=dn,
                                  preferred_element_type=jnp.float32))
    G32 = jnp.concatenate(chunks, axis=1)
    G = G32[0:16] + G32[16:32]
    f = f_ref[...]

    S2 = G[0:1]
    Sxb = G[1:2]
    Sxs = G[2:3]

    h = jnp.maximum(G[3:3 + _HID] + b1_ref[...], 0.0)
    p0 = jnp.sum(h * w2a_ref[...], axis=0, keepdims=True) + b2_ref[0:1]
    p1 = jnp.sum(h * w2b_ref[...], axis=0, keepdims=True) + b2_ref[1:2]

    a = base_ref[...] + slope_ref[...] * f
    mc = mcyt_ref[...]
    mx = jnp.maximum(jnp.maximum(a, mc), 0.0)
    lse = mx + jnp.log(jnp.exp(-mx) + jnp.exp(a - mx) + jnp.exp(mc - mx))
    sum_lse = jnp.sum(lse, axis=0, keepdims=True)

    m = m_ref[...]
    r = jax.lax.broadcasted_iota(jnp.int32, (1, _NUM_RES), 1)
    sp = jnp.maximum(m, 0.0) + jnp.log1p(jnp.exp(-jnp.abs(m)))
    c_non = jnp.sum(jnp.where((r % 4) != 0, sp, 0.0))

    logp = S2 + Sxb + Sxs * f - c_non - sum_lse
    out_ref[...] = jnp.concatenate([logp, p0, p1], axis=0)


def _hilo(Mf):
    hi = Mf.astype(jnp.bfloat16)
    lo = (Mf - hi.astype(jnp.float32)).astype(jnp.bfloat16)
    return jnp.concatenate([hi, lo], axis=0)


def kernel(dharma_output, fitness, mutation_rates, baseline_edits,
           slope_edits, W1, b1, W2, b2):
    B = dharma_output.shape[0]
    dt = jnp.transpose(dharma_output, (2, 1, 0))

    A = jnp.zeros((16, _NUM_RES), dtype=jnp.float32)
    A = A.at[1, _CYT].set(baseline_edits)
    A = A.at[2, _CYT].set(slope_edits)
    A = A.at[3:3 + _HID, _CYT].set(W1.T)
    Bm = jnp.zeros((16, _NUM_RES), dtype=jnp.float32)
    Bm = Bm.at[0].set(mutation_rates)

    f2 = fitness.reshape(1, B)
    m2 = mutation_rates.reshape(1, _NUM_RES)
    mcyt = mutation_rates[_CYT].reshape(_NCYT, 1)
    base = baseline_edits.reshape(_NCYT, 1)
    slope = slope_edits.reshape(_NCYT, 1)
    b1c = b1.reshape(_HID, 1)
    w2a = W2[:, 0].reshape(_HID, 1)
    w2b = W2[:, 1].reshape(_HID, 1)
    b2c = b2.reshape(2, 1)

    grid = B // _NB
    outT = pl.pallas_call(
        _body,
        grid=(grid,),
        in_specs=[spec for k in range(_NSPLIT) for spec in (
            pl.BlockSpec((1, _NUM_RES, _NC),
                         lambda i, k=k: (1, 0, i * _NSPLIT + k)),
            pl.BlockSpec((1, _NUM_RES, _NC),
                         lambda i, k=k: (2, 0, i * _NSPLIT + k)),
        )] + [
            pl.BlockSpec((1, _NB), lambda i: (0, i)),
            pl.BlockSpec((32, _NUM_RES), lambda i: (0, 0)),
            pl.BlockSpec((32, _NUM_RES), lambda i: (0, 0)),
            pl.BlockSpec((1, _NUM_RES), lambda i: (0, 0)),
            pl.BlockSpec((_NCYT, 1), lambda i: (0, 0)),
            pl.BlockSpec((_NCYT, 1), lambda i: (0, 0)),
            pl.BlockSpec((_NCYT, 1), lambda i: (0, 0)),
            pl.BlockSpec((_HID, 1), lambda i: (0, 0)),
            pl.BlockSpec((_HID, 1), lambda i: (0, 0)),
            pl.BlockSpec((_HID, 1), lambda i: (0, 0)),
            pl.BlockSpec((2, 1), lambda i: (0, 0)),
        ],
        out_specs=pl.BlockSpec((3, _NB), lambda i: (0, i)),
        out_shape=jax.ShapeDtypeStruct((3, B), jnp.float32),
    )(*([dt] * (2 * _NSPLIT)), f2, _hilo(A), _hilo(Bm), m2, mcyt, base, slope,
      b1c, w2a, w2b, b2c)
    return outT.T


# speedup vs baseline: 10.5263x; 10.5263x over previous
import numpy as np
import jax
import jax.numpy as jnp
from jax.experimental import pallas as pl

_NUM_RES = 190
_CYT = np.arange(0, _NUM_RES, 4)
_NCYT = len(_CYT)
_HID = 10
_NB = 4096
_NSPLIT = 4
_NC = _NB // _NSPLIT


def _body(*refs):
    d_refs = refs[:2 * _NSPLIT]
    (f_ref, A_ref, Bm_ref, m_ref, mcyt_ref, base_ref,
     slope_ref, b1_ref, w2a_ref, w2b_ref, b2_ref, out_ref) = refs[2 * _NSPLIT:]
    dn = (((1,), (0,)), ((), ()))
    chunks = []
    for k in range(_NSPLIT):
        d1 = d_refs[2 * k][0].astype(jnp.bfloat16)
        d2 = d_refs[2 * k + 1][0].astype(jnp.bfloat16)
        chunks.append(
            jax.lax.dot_general(A_ref[...], d1, dimension_numbers=dn,
                                preferred_element_type=jnp.float32)
            + jax.lax.dot_general(Bm_ref[...], d2, dimension_numbers=dn,
                                  preferred_element_type=jnp.float32))
    G32 = jnp.concatenate(chunks, axis=1)
    G = G32[0:16] + G32[16:32]
    f = f_ref[...]

    S2 = G[0:1]
    Sxb = G[1:2]
    Sxs = G[2:3]

    h = jnp.maximum(G[3:3 + _HID] + b1_ref[...], 0.0)
    p0 = jnp.sum(h * w2a_ref[...], axis=0, keepdims=True) + b2_ref[0:1]
    p1 = jnp.sum(h * w2b_ref[...], axis=0, keepdims=True) + b2_ref[1:2]

    a = base_ref[...] + slope_ref[...] * f
    mc = mcyt_ref[...]
    mx = jnp.maximum(jnp.maximum(a, mc), 0.0)
    lse = mx + jnp.log(jnp.exp(-mx) + jnp.exp(a - mx) + jnp.exp(mc - mx))
    sum_lse = jnp.sum(lse, axis=0, keepdims=True)

    m = m_ref[...]
    r = jax.lax.broadcasted_iota(jnp.int32, (1, _NUM_RES), 1)
    sp = jnp.maximum(m, 0.0) + jnp.log1p(jnp.exp(-jnp.abs(m)))
    c_non = jnp.sum(jnp.where((r % 4) != 0, sp, 0.0))

    logp = S2 + Sxb + Sxs * f - c_non - sum_lse
    out_ref[...] = jnp.concatenate([logp, p0, p1], axis=0)


def _hilo(Mf):
    hi = Mf.astype(jnp.bfloat16)
    lo = (Mf - hi.astype(jnp.float32)).astype(jnp.bfloat16)
    return jnp.concatenate([hi, lo], axis=0)


def kernel(dharma_output, fitness, mutation_rates, baseline_edits,
           slope_edits, W1, b1, W2, b2):
    B = dharma_output.shape[0]
    dt = jnp.transpose(dharma_output, (2, 1, 0))

    S = np.zeros((_NCYT, _NUM_RES), dtype=np.float32)
    S[np.arange(_NCYT), _CYT] = 1.0
    z = jnp.zeros((1, _NUM_RES), dtype=jnp.float32)
    cyt_rows = jnp.concatenate(
        [baseline_edits[None, :], slope_edits[None, :], W1.T], axis=0) @ S
    A = jnp.concatenate([z, cyt_rows, z, z, z, z], axis=0)
    Bm = jnp.concatenate([mutation_rates[None, :]] + [z] * 15, axis=0)

    f2 = fitness.reshape(1, B)
    m2 = mutation_rates.reshape(1, _NUM_RES)
    mcyt = mutation_rates[_CYT].reshape(_NCYT, 1)
    base = baseline_edits.reshape(_NCYT, 1)
    slope = slope_edits.reshape(_NCYT, 1)
    b1c = b1.reshape(_HID, 1)
    w2a = W2[:, 0].reshape(_HID, 1)
    w2b = W2[:, 1].reshape(_HID, 1)
    b2c = b2.reshape(2, 1)

    grid = B // _NB
    outT = pl.pallas_call(
        _body,
        grid=(grid,),
        in_specs=[spec for k in range(_NSPLIT) for spec in (
            pl.BlockSpec((1, _NUM_RES, _NC),
                         lambda i, k=k: (1, 0, i * _NSPLIT + k)),
            pl.BlockSpec((1, _NUM_RES, _NC),
                         lambda i, k=k: (2, 0, i * _NSPLIT + k)),
        )] + [
            pl.BlockSpec((1, _NB), lambda i: (0, i)),
            pl.BlockSpec((32, _NUM_RES), lambda i: (0, 0)),
            pl.BlockSpec((32, _NUM_RES), lambda i: (0, 0)),
            pl.BlockSpec((1, _NUM_RES), lambda i: (0, 0)),
            pl.BlockSpec((_NCYT, 1), lambda i: (0, 0)),
            pl.BlockSpec((_NCYT, 1), lambda i: (0, 0)),
            pl.BlockSpec((_NCYT, 1), lambda i: (0, 0)),
            pl.BlockSpec((_HID, 1), lambda i: (0, 0)),
            pl.BlockSpec((_HID, 1), lambda i: (0, 0)),
            pl.BlockSpec((_HID, 1), lambda i: (0, 0)),
            pl.BlockSpec((2, 1), lambda i: (0, 0)),
        ],
        out_specs=pl.BlockSpec((3, _NB), lambda i: (0, i)),
        out_shape=jax.ShapeDtypeStruct((3, B), jnp.float32),
    )(*([dt] * (2 * _NSPLIT)), f2, _hilo(A), _hilo(Bm), m2, mcyt, base, slope,
      b1c, w2a, w2b, b2c)
    return outT.T


# speedup vs baseline: 10.7298x; 1.0193x over previous
import numpy as np
import jax
import jax.numpy as jnp
from jax.experimental import pallas as pl

_NUM_RES = 190
_CYT = np.arange(0, _NUM_RES, 4)
_NCYT = len(_CYT)
_HID = 10
_NB = 4096


def _body(d1_ref, d2_ref, f_ref, C_ref, mcyt_ref, base_ref,
          slope_ref, b1_ref, w2a_ref, w2b_ref, b2_ref, out_ref):
    d1 = d1_ref[0].astype(jnp.bfloat16)
    d2 = d2_ref[0].astype(jnp.bfloat16)
    dn = (((1,), (0,)), ((), ()))
    C = C_ref[...]
    G = (jax.lax.dot_general(C[0:16], d1, dimension_numbers=dn,
                             preferred_element_type=jnp.float32)
         + jax.lax.dot_general(C[16:32], d2, dimension_numbers=dn,
                               preferred_element_type=jnp.float32)
         + jax.lax.dot_general(C[32:48], d1, dimension_numbers=dn,
                               preferred_element_type=jnp.float32)
         + jax.lax.dot_general(C[48:64], d2, dimension_numbers=dn,
                               preferred_element_type=jnp.float32))
    f = f_ref[...]

    S2 = G[0:1]
    Sxb = G[1:2]
    Sxs = G[2:3]

    h = jnp.maximum(G[3:3 + _HID] + b1_ref[...], 0.0)
    p0 = jnp.sum(h * w2a_ref[...], axis=0, keepdims=True) + b2_ref[0:1]
    p1 = jnp.sum(h * w2b_ref[...], axis=0, keepdims=True) + b2_ref[1:2]

    a = base_ref[...] + slope_ref[...] * f
    mc = mcyt_ref[...]
    mx = jnp.maximum(jnp.maximum(a, mc), 0.0)
    lse = mx + jnp.log(jnp.exp(-mx) + jnp.exp(a - mx) + jnp.exp(mc - mx))
    sum_lse = jnp.sum(lse, axis=0, keepdims=True)

    m = C[16:17]
    r = jax.lax.broadcasted_iota(jnp.int32, (1, _NUM_RES), 1)
    mf = m.astype(jnp.float32) + C[48:49].astype(jnp.float32)
    sp = jnp.maximum(mf, 0.0) + jnp.log1p(jnp.exp(-jnp.abs(mf)))
    c_non = jnp.sum(jnp.where((r % 4) != 0, sp, 0.0))

    logp = S2 + Sxb + Sxs * f - c_non - sum_lse
    out_ref[...] = jnp.concatenate([logp, p0, p1], axis=0)


def kernel(dharma_output, fitness, mutation_rates, baseline_edits,
           slope_edits, W1, b1, W2, b2):
    B = dharma_output.shape[0]
    dt = jnp.transpose(dharma_output, (2, 1, 0))

    S = np.zeros((_NCYT, _NUM_RES), dtype=np.float32)
    S[np.arange(_NCYT), _CYT] = 1.0
    z = jnp.zeros((1, _NUM_RES), dtype=jnp.float32)
    cyt_rows = jnp.concatenate(
        [baseline_edits[None, :], slope_edits[None, :], W1.T], axis=0) @ S
    Cf = jnp.concatenate(
        [z, cyt_rows, z, z, z, mutation_rates[None, :]] + [z] * 15, axis=0)
    C_hi = Cf.astype(jnp.bfloat16)
    C_lo = (Cf - C_hi.astype(jnp.float32)).astype(jnp.bfloat16)
    C = jnp.concatenate([C_hi, C_lo], axis=0)

    f2 = fitness.reshape(1, B)
    mcyt = mutation_rates[_CYT].reshape(_NCYT, 1)
    base = baseline_edits.reshape(_NCYT, 1)
    slope = slope_edits.reshape(_NCYT, 1)
    b1c = b1.reshape(_HID, 1)
    w2a = W2[:, 0].reshape(_HID, 1)
    w2b = W2[:, 1].reshape(_HID, 1)
    b2c = b2.reshape(2, 1)

    grid = B // _NB
    outT = pl.pallas_call(
        _body,
        grid=(grid,),
        in_specs=[
            pl.BlockSpec((1, _NUM_RES, _NB), lambda i: (1, 0, i)),
            pl.BlockSpec((1, _NUM_RES, _NB), lambda i: (2, 0, i)),
            pl.BlockSpec((1, _NB), lambda i: (0, i)),
            pl.BlockSpec((64, _NUM_RES), lambda i: (0, 0)),
            pl.BlockSpec((_NCYT, 1), lambda i: (0, 0)),
            pl.BlockSpec((_NCYT, 1), lambda i: (0, 0)),
            pl.BlockSpec((_NCYT, 1), lambda i: (0, 0)),
            pl.BlockSpec((_HID, 1), lambda i: (0, 0)),
            pl.BlockSpec((_HID, 1), lambda i: (0, 0)),
            pl.BlockSpec((_HID, 1), lambda i: (0, 0)),
            pl.BlockSpec((2, 1), lambda i: (0, 0)),
        ],
        out_specs=pl.BlockSpec((3, _NB), lambda i: (0, i)),
        out_shape=jax.ShapeDtypeStruct((3, B), jnp.float32),
    )(dt, dt, f2, C, mcyt, base, slope, b1c, w2a, w2b, b2c)
    return outT.T
